# Initial kernel scaffold; baseline (speedup 1.0000x reference)
import jax
import jax.numpy as jnp
from jax.experimental import pallas as pl


def kernel(x, node_types, edge_index, edge_types, W_type, W_out, b_out):
    raise NotImplementedError("write your pallas kernel here")


# speedup vs baseline: 3.9416x; 3.9416x over previous
import functools

import jax
import jax.numpy as jnp
from jax import lax
from jax.experimental import pallas as pl
from jax.experimental.pallas import tpu as pltpu
from jax.experimental.pallas import tpu_sc as plsc

_N = 10000
_D = 128
_T = 4
_C = 16

_TILES = 32
_B = 128
_CH = 79
_EP = _TILES * _CH * _B
_NP = 10240
_STRIPE = _NP // 16


def _s1_body(x_ref, nt_ref, w_ref, h_ref):
    xb = x_ref[...]
    nt = nt_ref[...]
    acc = jnp.zeros_like(h_ref)
    for t in range(_T):
        ht = jnp.dot(xb, w_ref[t], preferred_element_type=jnp.float32)
        acc = acc + jnp.where(nt == t, ht, 0.0)
    h_ref[...] = acc


def _s3_body(h_ref, a_ref, w_ref, b_ref, o_ref):
    h2 = jnp.maximum(h_ref[...] + a_ref[0] + a_ref[1], 0.0)
    o_ref[...] = jnp.dot(h2, w_ref[...], preferred_element_type=jnp.float32) + b_ref[...]


def _sc_body(h_hbm, src_hbm, dst_hbm, et_hbm, out_hbm,
             src_v, dst_v, et_v, rows_v, agg_sh, gsem):
    c = lax.axis_index("c")
    s = lax.axis_index("s")
    tid = c * 16 + s

    pltpu.sync_copy(src_hbm.at[tid], src_v)
    pltpu.sync_copy(dst_hbm.at[tid], dst_v)
    pltpu.sync_copy(et_hbm.at[tid], et_v)

    def _zrow(e, _):
        for r in range(_D // 16):
            rows_v[e, pl.ds(r * 16, 16)] = jnp.zeros((16,), jnp.float32)
        return 0

    lax.fori_loop(0, _B, _zrow, 0)
    base = s * _STRIPE
    for k in range(_STRIPE // _B):
        pltpu.sync_copy(rows_v, agg_sh.at[pl.ds(base + k * _B, _B)])
    plsc.subcore_barrier()

    def _chunk(j, _):
        pltpu.async_copy(h_hbm.at[src_v.at[j]], rows_v, gsem).wait()

        def _scale(g, _):
            etg = et_v[j, pl.ds(g * 16, 16)]
            for i in range(16):
                et = jnp.full((16,), etg[i])
                e = g * 16 + i
                for r in range(_D // 16):
                    sl = pl.ds(r * 16, 16)
                    rows_v[e, sl] = rows_v[e, sl] * et
            return 0

        lax.fori_loop(0, _B // 16, _scale, 0)
        pltpu.sync_copy(rows_v, agg_sh.at[dst_v.at[j]], add=True)
        return 0

    lax.fori_loop(0, _CH, _chunk, 0)
    plsc.subcore_barrier()

    pltpu.sync_copy(agg_sh.at[pl.ds(base, _STRIPE)],
                    out_hbm.at[c, pl.ds(base, _STRIPE)])


@functools.cache
def _get_sc_agg():
    return functools.partial(
        pl.kernel,
        out_type=jax.ShapeDtypeStruct((2, _NP, _D), jnp.float32),
        mesh=plsc.VectorSubcoreMesh(core_axis_name="c", subcore_axis_name="s"),
        scratch_types=[
            pltpu.VMEM((_CH, _B), jnp.int32),
            pltpu.VMEM((_CH, _B), jnp.int32),
            pltpu.VMEM((_CH, _B), jnp.float32),
            pltpu.VMEM((_B, _D), jnp.float32),
            pltpu.VMEM_SHARED((_NP, _D), jnp.float32),
            pltpu.SemaphoreType.DMA,
        ],
    )(_sc_body)


def kernel(x, node_types, edge_index, edge_types, W_type, W_out, b_out):
    n = x.shape[0]
    nb = 5
    blk = n // nb

    h = pl.pallas_call(
        _s1_body,
        grid=(nb,),
        in_specs=[
            pl.BlockSpec((blk, _D), lambda i: (i, 0)),
            pl.BlockSpec((blk, 1), lambda i: (i, 0)),
            pl.BlockSpec((_T, _D, _D), lambda i: (0, 0, 0)),
        ],
        out_specs=pl.BlockSpec((blk, _D), lambda i: (i, 0)),
        out_shape=jax.ShapeDtypeStruct((n, _D), jnp.float32),
    )(x, node_types.reshape(n, 1), W_type)

    e = edge_index.shape[1]
    pad = _EP - e
    src = jnp.pad(edge_index[0], (0, pad)).reshape(_TILES, _CH, _B)
    dst = jnp.pad(edge_index[1], (0, pad)).reshape(_TILES, _CH, _B)
    et = jnp.pad(edge_types, (0, pad)).reshape(_TILES, _CH, _B)

    agg = _get_sc_agg()(h, src, dst, et)

    out = pl.pallas_call(
        _s3_body,
        grid=(nb,),
        in_specs=[
            pl.BlockSpec((blk, _D), lambda i: (i, 0)),
            pl.BlockSpec((2, blk, _D), lambda i: (0, i, 0)),
            pl.BlockSpec((_D, _C), lambda i: (0, 0)),
            pl.BlockSpec((1, _C), lambda i: (0, 0)),
        ],
        out_specs=pl.BlockSpec((blk, _C), lambda i: (i, 0)),
        out_shape=jax.ShapeDtypeStruct((n, _C), jnp.float32),
    )(h, agg, W_out, b_out.reshape(1, _C))

    return out


# speedup vs baseline: 6.1292x; 1.5550x over previous
import functools

import jax
import jax.numpy as jnp
from jax import lax
from jax.experimental import pallas as pl
from jax.experimental.pallas import tpu as pltpu
from jax.experimental.pallas import tpu_sc as plsc

_N = 10000
_D = 128
_T = 4
_C = 16

_TILES = 32
_B = 112
_CH = 90
_EP = _TILES * _CH * _B
_G = 3
_NG = _CH // _G
_SG = _CH // (3 * _G)
_AGG_N = 10112
_STRIPE = _AGG_N // 16


def _s1_body(x_ref, nt_ref, w_ref, h_ref):
    xb = x_ref[...]
    nt = nt_ref[...]
    acc = jnp.zeros_like(h_ref)
    for t in range(_T):
        ht = jnp.dot(xb, w_ref[t], preferred_element_type=jnp.float32)
        acc = acc + jnp.where(nt == t, ht, 0.0)
    h_ref[...] = acc


def _s3_body(h_ref, a_ref, w_ref, b_ref, o_ref):
    h2 = jnp.maximum(h_ref[...] + a_ref[0] + a_ref[1], 0.0)
    o_ref[...] = jnp.dot(h2, w_ref[...], preferred_element_type=jnp.float32) + b_ref[...]


def _sc_body(h_hbm, pk_hbm, et_hbm, out_hbm, pk_v, et_v, rows_v, agg_sh,
             gsems, ssems, psems):
    c = lax.axis_index("c")
    s = lax.axis_index("s")
    tid = c * 16 + s

    def _zrow(e, _):
        for r in range(_D // 16):
            rows_v[0, e, pl.ds(r * 16, 16)] = jnp.zeros((16,), jnp.float32)
        return 0

    lax.fori_loop(0, _B, _zrow, 0)
    base = s * _STRIPE
    nfull, rem = _STRIPE // _B, _STRIPE % _B
    for k in range(nfull):
        pltpu.sync_copy(rows_v.at[0], agg_sh.at[pl.ds(base + k * _B, _B)])
    if rem:
        pltpu.sync_copy(rows_v.at[0, pl.ds(0, rem)],
                        agg_sh.at[pl.ds(base + nfull * _B, rem)])
    plsc.subcore_barrier()

    def issue_pk(g, slot):
        pltpu.async_copy(pk_hbm.at[tid, g], pk_v.at[slot], psems[slot])
        pltpu.async_copy(et_hbm.at[tid, g], et_v.at[slot], psems[slot])

    def wait_pk(slot):
        pltpu.make_async_copy(pk_hbm.at[tid, 0], pk_v.at[slot],
                              psems[slot]).wait()
        pltpu.make_async_copy(et_hbm.at[tid, 0], et_v.at[slot],
                              psems[slot]).wait()

    def issue_gather(slot, ci, b):
        pltpu.async_copy(h_hbm.at[pk_v.at[slot, 2 * ci]], rows_v.at[b],
                         gsems[b])

    def wait_gather(b):
        pltpu.make_async_copy(h_hbm.at[pk_v.at[0, 0]], rows_v.at[b],
                              gsems[b]).wait()

    def issue_scatter(slot, ci, b):
        pltpu.async_copy(rows_v.at[b], agg_sh.at[pk_v.at[slot, 2 * ci + 1]],
                         ssems[b], add=True)

    def wait_scatter(b):
        pltpu.make_async_copy(rows_v.at[b], agg_sh.at[pk_v.at[0, 1]],
                              ssems[b]).wait()

    def scale(slot, ci, b):
        def _sc16(gr, _):
            etg = et_v[slot, ci, pl.ds(gr * 16, 16)]
            for i in range(16):
                et = jnp.full((16,), etg[i])
                e = gr * 16 + i
                for r in range(_D // 16):
                    sl = pl.ds(r * 16, 16)
                    rows_v[b, e, sl] = rows_v[b, e, sl] * et
            return 0

        lax.fori_loop(0, _B // 16, _sc16, 0)

    pltpu.sync_copy(pk_hbm.at[tid, 0], pk_v.at[0])
    pltpu.sync_copy(et_hbm.at[tid, 0], et_v.at[0])
    issue_pk(1, 1)
    issue_gather(0, 0, 0)
    issue_gather(0, 1, 1)

    def body(sg, peel_first, peel_last):
        g0 = sg * 3
        for gg in range(3):
            for b in range(3):
                wait_gather(b)
                scale(gg, b, b)
                issue_scatter(gg, b, b)
                if not (peel_first and gg == 0 and b < 2):
                    wait_scatter((b + 1) % 3)
                if b == 0:
                    issue_gather(gg, 2, 2)
                elif b == 1:
                    if not (peel_last and gg == 2):
                        wait_pk((gg + 1) % 3)
                        issue_gather((gg + 1) % 3, 0, 0)
                else:
                    if not (peel_last and gg >= 1):
                        issue_pk(g0 + gg + 2, (gg + 2) % 3)
                    if not (peel_last and gg == 2):
                        issue_gather((gg + 1) % 3, 1, 1)

    body(0, True, _SG == 1)

    def _mid(sg, _):
        body(sg, False, False)
        return 0

    lax.fori_loop(1, _SG - 1, _mid, 0)
    if _SG > 1:
        body(_SG - 1, False, True)

    wait_scatter(1)
    wait_scatter(2)
    plsc.subcore_barrier()

    pltpu.sync_copy(agg_sh.at[pl.ds(base, _STRIPE)],
                    out_hbm.at[c, pl.ds(base, _STRIPE)])


@functools.cache
def _get_sc_agg():
    return functools.partial(
        pl.kernel,
        out_type=jax.ShapeDtypeStruct((2, _AGG_N, _D), jnp.float32),
        mesh=plsc.VectorSubcoreMesh(core_axis_name="c", subcore_axis_name="s"),
        scratch_types=[
            pltpu.VMEM((3, 2 * _G, _B), jnp.int32),
            pltpu.VMEM((3, _G, _B), jnp.float32),
            pltpu.VMEM((3, _B, _D), jnp.float32),
            pltpu.VMEM_SHARED((_AGG_N, _D), jnp.float32),
            [pltpu.SemaphoreType.DMA] * 3,
            [pltpu.SemaphoreType.DMA] * 3,
            [pltpu.SemaphoreType.DMA] * 3,
        ],
    )(_sc_body)


def kernel(x, node_types, edge_index, edge_types, W_type, W_out, b_out):
    n = x.shape[0]
    nb = 5
    blk = n // nb

    h = pl.pallas_call(
        _s1_body,
        grid=(nb,),
        in_specs=[
            pl.BlockSpec((blk, _D), lambda i: (i, 0)),
            pl.BlockSpec((blk, 1), lambda i: (i, 0)),
            pl.BlockSpec((_T, _D, _D), lambda i: (0, 0, 0)),
        ],
        out_specs=pl.BlockSpec((blk, _D), lambda i: (i, 0)),
        out_shape=jax.ShapeDtypeStruct((n, _D), jnp.float32),
    )(x, node_types.reshape(n, 1), W_type)

    e = edge_index.shape[1]
    pad = _EP - e
    src = jnp.pad(edge_index[0], (0, pad)).reshape(_TILES, _CH, _B)
    dst = jnp.pad(edge_index[1], (0, pad)).reshape(_TILES, _CH, _B)
    et3 = jnp.pad(edge_types, (0, pad)).reshape(_TILES, _NG, _G, _B)
    pk = jnp.stack([src, dst], axis=2)
    pk = pk.reshape(_TILES, _NG, 2 * _G, _B)

    agg = _get_sc_agg()(h, pk, et3)

    out = pl.pallas_call(
        _s3_body,
        grid=(nb,),
        in_specs=[
            pl.BlockSpec((blk, _D), lambda i: (i, 0)),
            pl.BlockSpec((2, blk, _D), lambda i: (0, i, 0)),
            pl.BlockSpec((_D, _C), lambda i: (0, 0)),
            pl.BlockSpec((1, _C), lambda i: (0, 0)),
        ],
        out_specs=pl.BlockSpec((blk, _C), lambda i: (i, 0)),
        out_shape=jax.ShapeDtypeStruct((n, _C), jnp.float32),
    )(h, agg, W_out, b_out.reshape(1, _C))

    return out
